# baseline (speedup 1.0000x reference)
import functools

import jax
import jax.numpy as jnp
from jax import lax
from jax.experimental import pallas as pl
from jax.experimental.pallas import tpu as pltpu
from jax.experimental.pallas import tpu_sc as plsc

B = 16384
HIST = 200
D = 32
OUT = 72

NC = 2
NS = 16
NW = NC * NS
PER_W = B // NW
CB = 8
NCHUNK = PER_W // CB
SCALE = 5.0 / HIST


def _body(uidx_hbm, gend_hbm, age_hbm, cidx_hbm, utbl_hbm, gtbl_hbm,
          atbl_hbm, ctbl_hbm, out_hbm,
          cidx_v, rows_v, uidx_v, gend_v, age_v, urows_v, grows_v, arows_v,
          out_v, semg0, semg1, semw0, semw1):
    semg = (semg0, semg1)
    semw = (semw0, semw1)
    wid = lax.axis_index("s") * NC + lax.axis_index("c")
    base0 = wid * PER_W

    pltpu.sync_copy(uidx_hbm.at[pl.ds(base0, PER_W)], uidx_v)
    pltpu.sync_copy(gend_hbm.at[pl.ds(base0, PER_W)], gend_v)
    pltpu.sync_copy(age_hbm.at[pl.ds(base0, PER_W)], age_v)

    def issue(c, ph):
        base = base0 + c * CB
        pltpu.sync_copy(cidx_hbm.at[pl.ds(base * HIST, CB * HIST)],
                        cidx_v.at[ph])
        pltpu.async_copy(ctbl_hbm.at[cidx_v.at[ph]], rows_v.at[ph], semg[ph])
        pltpu.async_copy(utbl_hbm.at[uidx_v.at[pl.ds(c * CB, CB)]],
                         urows_v.at[ph], semg[ph])
        pltpu.async_copy(gtbl_hbm.at[gend_v.at[pl.ds(c * CB, CB)]],
                         grows_v.at[ph], semg[ph])
        pltpu.async_copy(atbl_hbm.at[age_v.at[pl.ds(c * CB, CB)]],
                         arows_v.at[ph], semg[ph])

    def wait_gathers(ph):
        pltpu.make_async_copy(ctbl_hbm.at[pl.ds(0, CB * HIST)],
                              rows_v.at[ph], semg[ph]).wait()
        pltpu.make_async_copy(utbl_hbm.at[pl.ds(0, CB)],
                              urows_v.at[ph], semg[ph]).wait()
        pltpu.make_async_copy(gtbl_hbm.at[pl.ds(0, CB)],
                              grows_v.at[ph], semg[ph]).wait()
        pltpu.make_async_copy(atbl_hbm.at[pl.ds(0, CB)],
                              arows_v.at[ph], semg[ph]).wait()

    def drain_out(ph):
        pltpu.make_async_copy(out_v.at[ph],
                              out_hbm.at[pl.ds(0, CB * OUT)], semw[ph]).wait()

    def compute(c, ph):
        rows = rows_v.at[ph]
        out = out_v.at[ph]

        def pool(b, carry2):
            zero = jnp.zeros((16,), jnp.float32)

            @plsc.parallel_loop(0, HIST, step=4, unroll=2,
                                carry=(zero, zero, zero, zero))
            def red(h, accs):
                a0, a1, b0, b1 = accs
                r = b * HIST + h
                a0 = a0 + rows[r, pl.ds(0, 16)]
                a1 = a1 + rows[r, pl.ds(16, 16)]
                b0 = b0 + rows[r + 1, pl.ds(0, 16)]
                b1 = b1 + rows[r + 1, pl.ds(16, 16)]
                a0 = a0 + rows[r + 2, pl.ds(0, 16)]
                a1 = a1 + rows[r + 2, pl.ds(16, 16)]
                b0 = b0 + rows[r + 3, pl.ds(0, 16)]
                b1 = b1 + rows[r + 3, pl.ds(16, 16)]
                return a0, a1, b0, b1

            a0, a1, b0, b1 = red
            out[pl.ds(b * OUT, 16)] = urows_v[ph, b, pl.ds(0, 16)]
            out[pl.ds(b * OUT + 16, 16)] = urows_v[ph, b, pl.ds(16, 16)]
            out[pl.ds(b * OUT + 32, 16)] = grows_v[ph, b, pl.ds(0, 16)]
            out[pl.ds(b * OUT + 36, 16)] = arows_v[ph, b, pl.ds(0, 16)]
            out[pl.ds(b * OUT + 40, 16)] = (a0 + b0) * SCALE
            out[pl.ds(b * OUT + 56, 16)] = (a1 + b1) * SCALE
            return carry2

        lax.fori_loop(0, CB, pool, 0)
        base = base0 + c * CB
        pltpu.async_copy(out_v.at[ph], out_hbm.at[pl.ds(base * OUT, CB * OUT)],
                         semw[ph])

    issue(0, 0)

    def pair_body(p, carry):
        for ph in range(2):
            c = p * 2 + ph

            @pl.when(c + 1 < NCHUNK)
            def _():
                issue(c + 1, 1 - ph)

            wait_gathers(ph)

            @pl.when(c >= 2)
            def _():
                drain_out(ph)

            compute(c, ph)
        return carry

    lax.fori_loop(0, NCHUNK // 2, pair_body, 0)
    drain_out(0)
    drain_out(1)


@functools.lru_cache(maxsize=None)
def _build(interpret: bool = False):
    return functools.partial(
        pl.kernel,
        out_type=jax.ShapeDtypeStruct((B * OUT,), jnp.float32),
        mesh=plsc.VectorSubcoreMesh(core_axis_name="c", subcore_axis_name="s",
                                    num_cores=NC, num_subcores=NS),
        scratch_types=[
            pltpu.VMEM((2, CB * HIST,), jnp.int32),
            pltpu.VMEM((2, CB * HIST, D), jnp.float32),
            pltpu.VMEM((PER_W,), jnp.int32),
            pltpu.VMEM((PER_W,), jnp.int32),
            pltpu.VMEM((PER_W,), jnp.int32),
            pltpu.VMEM((2, CB, D), jnp.float32),
            pltpu.VMEM((2, CB, 16), jnp.float32),
            pltpu.VMEM((2, CB, 16), jnp.float32),
            pltpu.VMEM((2, CB * OUT), jnp.float32),
            pltpu.SemaphoreType.DMA,
            pltpu.SemaphoreType.DMA,
            pltpu.SemaphoreType.DMA,
            pltpu.SemaphoreType.DMA,
        ],
        compiler_params=pltpu.CompilerParams(use_tc_tiling_on_sc=False),
        interpret=interpret,
    )(lambda *refs: _body(*refs))


def kernel(user_idx, gender, age, context_idx, user_table, gender_table,
           age_table, context_table):
    gtbl = jnp.zeros((8, 16), jnp.float32).at[:3, :4].set(gender_table)
    atbl = jnp.zeros((104, 16), jnp.float32).at[:100, :4].set(age_table)
    out = _build()(
        user_idx.astype(jnp.int32),
        gender.astype(jnp.int32),
        age.astype(jnp.int32),
        context_idx.reshape(-1).astype(jnp.int32),
        user_table,
        gtbl,
        atbl,
        context_table,
    )
    return out.reshape(B, OUT)


# speedup vs baseline: 1.0905x; 1.0905x over previous
import functools

import jax
import jax.numpy as jnp
from jax import lax
from jax.experimental import pallas as pl
from jax.experimental.pallas import tpu as pltpu
from jax.experimental.pallas import tpu_sc as plsc

B = 16384
HIST = 200
D = 32
ROW_A = 40

NC = 2
NS = 16
NW = NC * NS
PER_W = B // NW
CB = 8
NCHUNK = PER_W // CB
SCALE = 5.0 / HIST

_MESH = dict(core_axis_name="c", subcore_axis_name="s",
             num_cores=NC, num_subcores=NS)


def _body_a(gend_hbm, age_hbm, cidx_hbm, gtbl_hbm, atbl_hbm, ctbl_hbm,
            out_hbm,
            cidx_v, rows_v, gend_v, age_v, grows_v, arows_v, out_v,
            semg0, semg1, semw0, semw1):
    semg = (semg0, semg1)
    semw = (semw0, semw1)
    wid = lax.axis_index("s") * NC + lax.axis_index("c")
    base0 = wid * PER_W

    pltpu.sync_copy(gend_hbm.at[pl.ds(base0, PER_W)], gend_v)
    pltpu.sync_copy(age_hbm.at[pl.ds(base0, PER_W)], age_v)

    def issue(c, ph):
        base = base0 + c * CB
        pltpu.sync_copy(cidx_hbm.at[pl.ds(base * HIST, CB * HIST)],
                        cidx_v.at[ph])
        pltpu.async_copy(ctbl_hbm.at[cidx_v.at[ph]], rows_v.at[ph], semg[ph])
        pltpu.async_copy(gtbl_hbm.at[gend_v.at[pl.ds(c * CB, CB)]],
                         grows_v.at[ph], semg[ph])
        pltpu.async_copy(atbl_hbm.at[age_v.at[pl.ds(c * CB, CB)]],
                         arows_v.at[ph], semg[ph])

    def wait_gathers(ph):
        pltpu.make_async_copy(ctbl_hbm.at[pl.ds(0, CB * HIST)],
                              rows_v.at[ph], semg[ph]).wait()
        pltpu.make_async_copy(gtbl_hbm.at[pl.ds(0, CB)],
                              grows_v.at[ph], semg[ph]).wait()
        pltpu.make_async_copy(atbl_hbm.at[pl.ds(0, CB)],
                              arows_v.at[ph], semg[ph]).wait()

    def drain_out(ph):
        pltpu.make_async_copy(out_v.at[ph],
                              out_hbm.at[pl.ds(0, CB * ROW_A)],
                              semw[ph]).wait()

    def compute(c, ph):
        rows = rows_v.at[ph]
        out = out_v.at[ph]

        def pool(b, carry2):
            zero = jnp.zeros((16,), jnp.float32)

            @plsc.parallel_loop(0, HIST, step=4, unroll=2,
                                carry=(zero, zero, zero, zero))
            def red(h, accs):
                a0, a1, b0, b1 = accs
                r = b * HIST + h
                a0 = a0 + rows[r, pl.ds(0, 16)]
                a1 = a1 + rows[r, pl.ds(16, 16)]
                b0 = b0 + rows[r + 1, pl.ds(0, 16)]
                b1 = b1 + rows[r + 1, pl.ds(16, 16)]
                a0 = a0 + rows[r + 2, pl.ds(0, 16)]
                a1 = a1 + rows[r + 2, pl.ds(16, 16)]
                b0 = b0 + rows[r + 3, pl.ds(0, 16)]
                b1 = b1 + rows[r + 3, pl.ds(16, 16)]
                return a0, a1, b0, b1

            a0, a1, b0, b1 = red
            out[pl.ds(b * ROW_A, 16)] = grows_v[ph, b, pl.ds(0, 16)]
            out[pl.ds(b * ROW_A + 4, 16)] = arows_v[ph, b, pl.ds(0, 16)]
            out[pl.ds(b * ROW_A + 8, 16)] = (a0 + b0) * SCALE
            out[pl.ds(b * ROW_A + 24, 16)] = (a1 + b1) * SCALE
            return carry2

        lax.fori_loop(0, CB, pool, 0)
        base = base0 + c * CB
        pltpu.async_copy(out_v.at[ph],
                         out_hbm.at[pl.ds(base * ROW_A, CB * ROW_A)],
                         semw[ph])

    issue(0, 0)

    def pair_body(p, carry):
        for ph in range(2):
            c = p * 2 + ph

            @pl.when(c + 1 < NCHUNK)
            def _():
                issue(c + 1, 1 - ph)

            wait_gathers(ph)

            @pl.when(c >= 2)
            def _():
                drain_out(ph)

            compute(c, ph)
        return carry

    lax.fori_loop(0, NCHUNK // 2, pair_body, 0)
    drain_out(0)
    drain_out(1)


def _body_b(uidx_hbm, utbl_hbm, out_hbm, uidx_v, urows_v, sem):
    wid = lax.axis_index("s") * NC + lax.axis_index("c")
    base0 = wid * PER_W
    pltpu.sync_copy(uidx_hbm.at[pl.ds(base0, PER_W)], uidx_v)
    pltpu.async_copy(utbl_hbm.at[uidx_v], urows_v, sem).wait()
    pltpu.sync_copy(urows_v, out_hbm.at[pl.ds(base0, PER_W)])


@functools.lru_cache(maxsize=None)
def _build(interpret: bool = False):
    ka = functools.partial(
        pl.kernel,
        out_type=jax.ShapeDtypeStruct((B * ROW_A,), jnp.float32),
        mesh=plsc.VectorSubcoreMesh(**_MESH),
        scratch_types=[
            pltpu.VMEM((2, CB * HIST,), jnp.int32),
            pltpu.VMEM((2, CB * HIST, D), jnp.float32),
            pltpu.VMEM((PER_W,), jnp.int32),
            pltpu.VMEM((PER_W,), jnp.int32),
            pltpu.VMEM((2, CB, 16), jnp.float32),
            pltpu.VMEM((2, CB, 16), jnp.float32),
            pltpu.VMEM((2, CB * ROW_A), jnp.float32),
            pltpu.SemaphoreType.DMA,
            pltpu.SemaphoreType.DMA,
            pltpu.SemaphoreType.DMA,
            pltpu.SemaphoreType.DMA,
        ],
        compiler_params=pltpu.CompilerParams(use_tc_tiling_on_sc=False),
        interpret=interpret,
    )(lambda *refs: _body_a(*refs))

    kb = functools.partial(
        pl.kernel,
        out_type=jax.ShapeDtypeStruct((B, D), jnp.float32),
        mesh=plsc.VectorSubcoreMesh(**_MESH),
        scratch_types=[
            pltpu.VMEM((PER_W,), jnp.int32),
            pltpu.VMEM((PER_W, D), jnp.float32),
            pltpu.SemaphoreType.DMA,
        ],
        compiler_params=pltpu.CompilerParams(use_tc_tiling_on_sc=False),
        interpret=interpret,
    )(lambda *refs: _body_b(*refs))
    return ka, kb


def kernel(user_idx, gender, age, context_idx, user_table, gender_table,
           age_table, context_table):
    ka, kb = _build()
    gtbl = jnp.zeros((8, 16), jnp.float32).at[:3, :4].set(gender_table)
    atbl = jnp.zeros((104, 16), jnp.float32).at[:100, :4].set(age_table)
    rest = ka(
        gender.astype(jnp.int32),
        age.astype(jnp.int32),
        context_idx.reshape(-1).astype(jnp.int32),
        gtbl,
        atbl,
        context_table,
    ).reshape(B, ROW_A)
    u = kb(user_idx.astype(jnp.int32), user_table)
    return jnp.concatenate([u, rest], axis=-1)


# speedup vs baseline: 1.3248x; 1.2149x over previous
import functools

import jax
import jax.numpy as jnp
from jax import lax
from jax.experimental import pallas as pl
from jax.experimental.pallas import tpu as pltpu
from jax.experimental.pallas import tpu_sc as plsc

B = 16384
HIST = 200
D = 32
ROW_A = 40

NC = 2
NS = 16
NW = NC * NS
PER_W = B // NW
CB = 8
NCHUNK = PER_W // CB
SCALE = 5.0 / HIST

_MESH = dict(core_axis_name="c", subcore_axis_name="s",
             num_cores=NC, num_subcores=NS)


def _body_a(gend_hbm, age_hbm, cidx_hbm, gtbl_hbm, atbl_hbm, ctbl_hbm,
            out_hbm,
            cidx_v, rows_v, gend_v, age_v, grows_v, arows_v, out_v,
            semg0, semg1, semw0, semw1):
    semg = (semg0, semg1)
    semw = (semw0, semw1)
    wid = lax.axis_index("s") * NC + lax.axis_index("c")
    base0 = wid * PER_W

    pltpu.sync_copy(gend_hbm.at[pl.ds(base0, PER_W)], gend_v)
    pltpu.sync_copy(age_hbm.at[pl.ds(base0, PER_W)], age_v)

    def issue(c, ph):
        base = base0 + c * CB
        pltpu.sync_copy(cidx_hbm.at[pl.ds(base * HIST, CB * HIST)],
                        cidx_v.at[ph])
        pltpu.async_copy(ctbl_hbm.at[cidx_v.at[ph]], rows_v.at[ph], semg[ph])
        pltpu.async_copy(gtbl_hbm.at[gend_v.at[pl.ds(c * CB, CB)]],
                         grows_v.at[ph], semg[ph])
        pltpu.async_copy(atbl_hbm.at[age_v.at[pl.ds(c * CB, CB)]],
                         arows_v.at[ph], semg[ph])

    def wait_gathers(ph):
        pltpu.make_async_copy(ctbl_hbm.at[pl.ds(0, CB * HIST)],
                              rows_v.at[ph], semg[ph]).wait()
        pltpu.make_async_copy(gtbl_hbm.at[pl.ds(0, CB)],
                              grows_v.at[ph], semg[ph]).wait()
        pltpu.make_async_copy(atbl_hbm.at[pl.ds(0, CB)],
                              arows_v.at[ph], semg[ph]).wait()

    def drain_out(ph):
        pltpu.make_async_copy(out_v.at[ph],
                              out_hbm.at[pl.ds(0, CB * ROW_A)],
                              semw[ph]).wait()

    def compute(c, ph):
        rows = rows_v.at[ph]
        out = out_v.at[ph]

        def pool(b, carry2):
            zero = jnp.zeros((16,), jnp.float32)

            @plsc.parallel_loop(0, HIST, step=4, unroll=2,
                                carry=(zero, zero, zero, zero))
            def red(h, accs):
                a0, a1, b0, b1 = accs
                r = b * HIST + h
                a0 = a0 + rows[r, pl.ds(0, 16)]
                a1 = a1 + rows[r, pl.ds(16, 16)]
                b0 = b0 + rows[r + 1, pl.ds(0, 16)]
                b1 = b1 + rows[r + 1, pl.ds(16, 16)]
                a0 = a0 + rows[r + 2, pl.ds(0, 16)]
                a1 = a1 + rows[r + 2, pl.ds(16, 16)]
                b0 = b0 + rows[r + 3, pl.ds(0, 16)]
                b1 = b1 + rows[r + 3, pl.ds(16, 16)]
                return a0, a1, b0, b1

            a0, a1, b0, b1 = red
            out[pl.ds(b * ROW_A, 16)] = grows_v[ph, b, pl.ds(0, 16)]
            out[pl.ds(b * ROW_A + 4, 16)] = arows_v[ph, b, pl.ds(0, 16)]
            out[pl.ds(b * ROW_A + 8, 16)] = (a0 + b0) * SCALE
            out[pl.ds(b * ROW_A + 24, 16)] = (a1 + b1) * SCALE
            return carry2

        lax.fori_loop(0, CB, pool, 0)
        base = base0 + c * CB
        pltpu.async_copy(out_v.at[ph],
                         out_hbm.at[pl.ds(base * ROW_A, CB * ROW_A)],
                         semw[ph])

    issue(0, 0)

    def pair_body(p, carry):
        for ph in range(2):
            c = p * 2 + ph

            @pl.when(c + 1 < NCHUNK)
            def _():
                issue(c + 1, 1 - ph)

            wait_gathers(ph)

            @pl.when(c >= 2)
            def _():
                drain_out(ph)

            compute(c, ph)
        return carry

    lax.fori_loop(0, NCHUNK // 2, pair_body, 0)
    drain_out(0)
    drain_out(1)


GRP = 16


def _body_b(uidx_hbm, utbl_hbm, out_hbm, uidx_v, slab_v, urows_v, sem):
    wid = lax.axis_index("s") * NC + lax.axis_index("c")
    base0 = wid * PER_W
    pltpu.sync_copy(uidx_hbm.at[pl.ds(base0, PER_W)], uidx_v)
    iota = lax.iota(jnp.int32, 16)

    def group(g, carry):
        idx16 = uidx_v[pl.ds(g * GRP, GRP)]
        rs = []
        for j in range(GRP):
            r = jnp.max(jnp.where(iota == j, idx16, 0))
            rs.append(r)
            r8 = pl.multiple_of((r // 8) * 8, 8)
            pltpu.async_copy(utbl_hbm.at[pl.ds(r8, 8), :], slab_v.at[j], sem)
        for j in range(GRP):
            i = g * GRP + j
            pltpu.make_async_copy(utbl_hbm.at[pl.ds(0, 8), :],
                                  slab_v.at[j], sem).wait()
            k = rs[j] - (rs[j] // 8) * 8
            urows_v[i, pl.ds(0, 16)] = slab_v[j, k, pl.ds(0, 16)]
            urows_v[i, pl.ds(16, 16)] = slab_v[j, k, pl.ds(16, 16)]
        return carry

    lax.fori_loop(0, PER_W // GRP, group, 0)
    pltpu.sync_copy(urows_v, out_hbm.at[pl.ds(base0, PER_W)])


@functools.lru_cache(maxsize=None)
def _build(interpret: bool = False):
    ka = functools.partial(
        pl.kernel,
        out_type=jax.ShapeDtypeStruct((B * ROW_A,), jnp.float32),
        mesh=plsc.VectorSubcoreMesh(**_MESH),
        scratch_types=[
            pltpu.VMEM((2, CB * HIST,), jnp.int32),
            pltpu.VMEM((2, CB * HIST, D), jnp.float32),
            pltpu.VMEM((PER_W,), jnp.int32),
            pltpu.VMEM((PER_W,), jnp.int32),
            pltpu.VMEM((2, CB, 16), jnp.float32),
            pltpu.VMEM((2, CB, 16), jnp.float32),
            pltpu.VMEM((2, CB * ROW_A), jnp.float32),
            pltpu.SemaphoreType.DMA,
            pltpu.SemaphoreType.DMA,
            pltpu.SemaphoreType.DMA,
            pltpu.SemaphoreType.DMA,
        ],
        compiler_params=pltpu.CompilerParams(use_tc_tiling_on_sc=False),
        interpret=interpret,
    )(lambda *refs: _body_a(*refs))

    kb = functools.partial(
        pl.kernel,
        out_type=jax.ShapeDtypeStruct((B, D), jnp.float32),
        mesh=plsc.VectorSubcoreMesh(**_MESH),
        scratch_types=[
            pltpu.VMEM((PER_W,), jnp.int32),
            pltpu.VMEM((GRP, 8, D), jnp.float32),
            pltpu.VMEM((PER_W, D), jnp.float32),
            pltpu.SemaphoreType.DMA,
        ],
        compiler_params=pltpu.CompilerParams(use_tc_tiling_on_sc=True,
                                             needs_layout_passes=False),
        interpret=interpret,
    )(lambda *refs: _body_b(*refs))
    return ka, kb


def kernel(user_idx, gender, age, context_idx, user_table, gender_table,
           age_table, context_table):
    ka, kb = _build()
    gtbl = jnp.zeros((8, 16), jnp.float32).at[:3, :4].set(gender_table)
    atbl = jnp.zeros((104, 16), jnp.float32).at[:100, :4].set(age_table)
    rest = ka(
        gender.astype(jnp.int32),
        age.astype(jnp.int32),
        context_idx.reshape(-1).astype(jnp.int32),
        gtbl,
        atbl,
        context_table,
    ).reshape(B, ROW_A)
    u = kb(user_idx.astype(jnp.int32), user_table)
    return jnp.concatenate([u, rest], axis=-1)


# speedup vs baseline: 1.7161x; 1.2953x over previous
import functools

import jax
import jax.numpy as jnp
from jax import lax
from jax.experimental import pallas as pl
from jax.experimental.pallas import tpu as pltpu
from jax.experimental.pallas import tpu_sc as plsc

B = 16384
HIST = 200
D = 32
ROW_A = 40

NC = 2
NS = 16
NW = NC * NS
PER_W = B // NW
CB = 8
NCHUNK = PER_W // CB
SCALE = 5.0 / HIST

_MESH = dict(core_axis_name="c", subcore_axis_name="s",
             num_cores=NC, num_subcores=NS)


def _body_a(gend_hbm, age_hbm, cidx_hbm, gtbl_hbm, atbl_hbm, ctbl_hbm,
            out_hbm,
            cidx_v, rows_v, gend_v, age_v, grows_v, arows_v, out_v,
            semg0, semg1, semw0, semw1):
    semg = (semg0, semg1)
    semw = (semw0, semw1)
    wid = lax.axis_index("s") * NC + lax.axis_index("c")
    base0 = wid * PER_W

    pltpu.sync_copy(gend_hbm.at[pl.ds(base0, PER_W)], gend_v)
    pltpu.sync_copy(age_hbm.at[pl.ds(base0, PER_W)], age_v)

    def issue(c, ph):
        base = base0 + c * CB
        pltpu.sync_copy(cidx_hbm.at[pl.ds(base * HIST, CB * HIST)],
                        cidx_v.at[ph])
        pltpu.async_copy(ctbl_hbm.at[cidx_v.at[ph]], rows_v.at[ph], semg[ph])
        pltpu.async_copy(gtbl_hbm.at[gend_v.at[pl.ds(c * CB, CB)]],
                         grows_v.at[ph], semg[ph])
        pltpu.async_copy(atbl_hbm.at[age_v.at[pl.ds(c * CB, CB)]],
                         arows_v.at[ph], semg[ph])

    def wait_gathers(ph):
        pltpu.make_async_copy(ctbl_hbm.at[pl.ds(0, CB * HIST)],
                              rows_v.at[ph], semg[ph]).wait()
        pltpu.make_async_copy(gtbl_hbm.at[pl.ds(0, CB)],
                              grows_v.at[ph], semg[ph]).wait()
        pltpu.make_async_copy(atbl_hbm.at[pl.ds(0, CB)],
                              arows_v.at[ph], semg[ph]).wait()

    def drain_out(ph):
        pltpu.make_async_copy(out_v.at[ph],
                              out_hbm.at[pl.ds(0, CB * ROW_A)],
                              semw[ph]).wait()

    def compute(c, ph):
        rows = rows_v.at[ph]
        out = out_v.at[ph]

        def pool(b, carry2):
            zero = jnp.zeros((16,), jnp.float32)

            @plsc.parallel_loop(0, HIST, step=4, unroll=2,
                                carry=(zero, zero, zero, zero))
            def red(h, accs):
                a0, a1, b0, b1 = accs
                r = b * HIST + h
                a0 = a0 + rows[r, pl.ds(0, 16)]
                a1 = a1 + rows[r, pl.ds(16, 16)]
                b0 = b0 + rows[r + 1, pl.ds(0, 16)]
                b1 = b1 + rows[r + 1, pl.ds(16, 16)]
                a0 = a0 + rows[r + 2, pl.ds(0, 16)]
                a1 = a1 + rows[r + 2, pl.ds(16, 16)]
                b0 = b0 + rows[r + 3, pl.ds(0, 16)]
                b1 = b1 + rows[r + 3, pl.ds(16, 16)]
                return a0, a1, b0, b1

            a0, a1, b0, b1 = red
            out[pl.ds(b * ROW_A, 16)] = grows_v[ph, b, pl.ds(0, 16)]
            out[pl.ds(b * ROW_A + 4, 16)] = arows_v[ph, b, pl.ds(0, 16)]
            out[pl.ds(b * ROW_A + 8, 16)] = (a0 + b0) * SCALE
            out[pl.ds(b * ROW_A + 24, 16)] = (a1 + b1) * SCALE
            return carry2

        lax.fori_loop(0, CB, pool, 0)
        base = base0 + c * CB
        pltpu.async_copy(out_v.at[ph],
                         out_hbm.at[pl.ds(base * ROW_A, CB * ROW_A)],
                         semw[ph])

    issue(0, 0)

    def pair_body(p, carry):
        for ph in range(2):
            c = p * 2 + ph

            @pl.when(c + 1 < NCHUNK)
            def _():
                issue(c + 1, 1 - ph)

            wait_gathers(ph)

            @pl.when(c >= 2)
            def _():
                drain_out(ph)

            compute(c, ph)
        return carry

    lax.fori_loop(0, NCHUNK // 2, pair_body, 0)
    drain_out(0)
    drain_out(1)


GRP = 8
NGRP = PER_W // GRP


def _body_b(uidx_hbm, utt_hbm, out_hbm, uidx_v, slab_v, urow_v,
            semg0, semg1, semw0, semw1):
    semg = (semg0, semg1)
    semw = (semw0, semw1)
    wid = lax.axis_index("s") * NC + lax.axis_index("c")
    base0 = wid * PER_W
    pltpu.sync_copy(uidx_hbm.at[pl.ds(base0, PER_W)],
                    uidx_v.at[pl.ds(0, PER_W)])
    iota = lax.iota(jnp.int32, 16)
    fdiv = jnp.right_shift(iota, 3)
    fmod = jnp.bitwise_and(iota, 7)

    def scalars(g, j):
        idx16 = uidx_v[pl.ds(g * GRP, 16)]
        r = jnp.max(jnp.where(iota == j, idx16, 0))
        c128 = pl.multiple_of((r // 128) * 128, 128)
        return r, c128

    def issue(g, ph):
        for j in range(GRP):
            r, c128 = scalars(g, j)
            for t in range(4):
                pltpu.async_copy(
                    utt_hbm.at[pl.ds(t * 8, 8), pl.ds(c128, 128)],
                    slab_v.at[ph, j, t], semg[ph])

    def drain_extract(g, ph):
        for j in range(GRP):
            for t in range(4):
                pltpu.make_async_copy(
                    utt_hbm.at[pl.ds(0, 8), pl.ds(0, 128)],
                    slab_v.at[ph, j, t], semg[ph]).wait()
            r, c128 = scalars(g, j)
            col = (r - c128) + jnp.zeros((16,), jnp.int32)
            phv = jnp.full((16,), ph, jnp.int32)
            jv = jnp.full((16,), j, jnp.int32)
            v0 = plsc.load_gather(slab_v, [phv, jv, fdiv, fmod, col])
            v1 = plsc.load_gather(slab_v, [phv, jv, 2 + fdiv, fmod, col])
            urow_v[ph, j, pl.ds(0, 16)] = v0
            urow_v[ph, j, pl.ds(16, 16)] = v1
        pltpu.async_copy(urow_v.at[ph],
                         out_hbm.at[pl.ds(base0 + g * GRP, GRP)], semw[ph])

    def drain_write(ph):
        pltpu.make_async_copy(urow_v.at[ph], out_hbm.at[pl.ds(0, GRP)],
                              semw[ph]).wait()

    issue(0, 0)

    def pair_body(p, carry):
        for ph in range(2):
            g = p * 2 + ph

            @pl.when(g + 1 < NGRP)
            def _():
                issue(g + 1, 1 - ph)

            @pl.when(g >= 2)
            def _():
                drain_write(ph)

            drain_extract(g, ph)
        return carry

    lax.fori_loop(0, NGRP // 2, pair_body, 0)
    drain_write(0)
    drain_write(1)


@functools.lru_cache(maxsize=None)
def _build(interpret: bool = False):
    ka = functools.partial(
        pl.kernel,
        out_type=jax.ShapeDtypeStruct((B * ROW_A,), jnp.float32),
        mesh=plsc.VectorSubcoreMesh(**_MESH),
        scratch_types=[
            pltpu.VMEM((2, CB * HIST,), jnp.int32),
            pltpu.VMEM((2, CB * HIST, D), jnp.float32),
            pltpu.VMEM((PER_W,), jnp.int32),
            pltpu.VMEM((PER_W,), jnp.int32),
            pltpu.VMEM((2, CB, 16), jnp.float32),
            pltpu.VMEM((2, CB, 16), jnp.float32),
            pltpu.VMEM((2, CB * ROW_A), jnp.float32),
            pltpu.SemaphoreType.DMA,
            pltpu.SemaphoreType.DMA,
            pltpu.SemaphoreType.DMA,
            pltpu.SemaphoreType.DMA,
        ],
        compiler_params=pltpu.CompilerParams(use_tc_tiling_on_sc=False),
        interpret=interpret,
    )(lambda *refs: _body_a(*refs))

    kb = functools.partial(
        pl.kernel,
        out_type=jax.ShapeDtypeStruct((B, D), jnp.float32),
        mesh=plsc.VectorSubcoreMesh(**_MESH),
        scratch_types=[
            pltpu.VMEM((PER_W + 16,), jnp.int32),
            pltpu.VMEM((2, GRP, 4, 8, 128), jnp.float32),
            pltpu.VMEM((2, GRP, D), jnp.float32),
            pltpu.SemaphoreType.DMA,
            pltpu.SemaphoreType.DMA,
            pltpu.SemaphoreType.DMA,
            pltpu.SemaphoreType.DMA,
        ],
        compiler_params=pltpu.CompilerParams(use_tc_tiling_on_sc=True,
                                             needs_layout_passes=False),
        interpret=interpret,
    )(lambda *refs: _body_b(*refs))
    return ka, kb


def kernel(user_idx, gender, age, context_idx, user_table, gender_table,
           age_table, context_table):
    ka, kb = _build()
    gtbl = jnp.zeros((8, 16), jnp.float32).at[:3, :4].set(gender_table)
    atbl = jnp.zeros((104, 16), jnp.float32).at[:100, :4].set(age_table)
    rest = ka(
        gender.astype(jnp.int32),
        age.astype(jnp.int32),
        context_idx.reshape(-1).astype(jnp.int32),
        gtbl,
        atbl,
        context_table,
    ).reshape(B, ROW_A)
    u = kb(user_idx.astype(jnp.int32), user_table.T)
    return jnp.concatenate([u, rest], axis=-1)
